# baseline (speedup 1.0000x reference)
import functools

import jax
import jax.numpy as jnp
from jax.experimental import pallas as pl
from jax.experimental.pallas import tpu as pltpu


def _hnhn_block(x0_ref, b_ref, w0_ref, w1_ref, b0_ref, b1_ref,
                x0_out_ref, x1_out_ref, h_ref, *, e_total):
    j = pl.program_id(0)
    nj = pl.num_programs(0)
    ej = x1_out_ref.shape[0]

    @pl.when(j == 0)
    def _():
        h_ref[...] = jnp.dot(x0_ref[...], w0_ref[...],
                             preferred_element_type=jnp.float32
                             ).astype(jnp.bfloat16)

    b_blk = b_ref[...].astype(jnp.bfloat16)

    x1 = jax.lax.dot_general(b_blk, h_ref[...], (((0,), (0,)), ((), ())),
                             preferred_element_type=jnp.float32)
    x1 = x1 + b0_ref[...]
    valid = e_total - j * ej
    row_ids = jax.lax.broadcasted_iota(jnp.int32, x1.shape, 0)
    x1 = jnp.where(row_ids < valid, x1, 0.0)
    x1_out_ref[...] = jnp.maximum(x1, 0.0)

    y = jnp.dot(x1, w1_ref[...],
                preferred_element_type=jnp.float32).astype(jnp.bfloat16)
    lane_ids = jax.lax.broadcasted_iota(jnp.int32, b_blk.shape, 1)
    b_masked = jnp.where(lane_ids < valid, b_blk, jnp.bfloat16(0.0))
    contrib = jax.lax.dot_general(b_masked, y, (((1,), (0,)), ((), ())),
                                  preferred_element_type=jnp.float32)

    @pl.when(j == 0)
    def _():
        x0_out_ref[...] = contrib

    @pl.when(j > 0)
    def _():
        x0_out_ref[...] += contrib

    @pl.when(j == nj - 1)
    def _():
        x0_out_ref[...] = jnp.maximum(x0_out_ref[...] + b1_ref[...], 0.0)


def kernel(x_0, incidence_1, W0, W1, bias_0_to_1, bias_1_to_0):
    n, d_in = x_0.shape
    e = incidence_1.shape[1]
    d = W0.shape[1]

    ej = min(384, ((e + 127) // 128) * 128)
    grid = ((e + ej - 1) // ej,)

    out0, out1 = pl.pallas_call(
        functools.partial(_hnhn_block, e_total=e),
        grid=grid,
        in_specs=[
            pl.BlockSpec((n, d_in), lambda j: (0, 0)),
            pl.BlockSpec((n, ej), lambda j: (0, j)),
            pl.BlockSpec((d_in, d), lambda j: (0, 0)),
            pl.BlockSpec((d, d), lambda j: (0, 0)),
            pl.BlockSpec((1, d), lambda j: (0, 0)),
            pl.BlockSpec((1, d), lambda j: (0, 0)),
        ],
        out_specs=[
            pl.BlockSpec((n, d), lambda j: (0, 0)),
            pl.BlockSpec((ej, d), lambda j: (j, 0)),
        ],
        out_shape=[
            jax.ShapeDtypeStruct((n, d), jnp.float32),
            jax.ShapeDtypeStruct((e, d), jnp.float32),
        ],
        scratch_shapes=[pltpu.VMEM((n, d), jnp.bfloat16)],
        compiler_params=pltpu.CompilerParams(
            dimension_semantics=("arbitrary",),
            vmem_limit_bytes=64 * 1024 * 1024,
        ),
    )(x_0, incidence_1, W0, W1, bias_0_to_1, bias_1_to_0)
    return (out0, out1)


# speedup vs baseline: 1.7332x; 1.7332x over previous
import functools

import jax
import jax.numpy as jnp
from jax.experimental import pallas as pl
from jax.experimental.pallas import tpu as pltpu


def _hnhn_block(x0_ref, b_ref, w0_ref, w1_ref, b0_ref, b1_ref,
                x0_out_ref, x1_out_ref, ht_ref, *, e_total):
    j = pl.program_id(0)
    nj = pl.num_programs(0)
    ej = x1_out_ref.shape[0]

    @pl.when(j == 0)
    def _():
        h = jnp.dot(x0_ref[...], w0_ref[...],
                    preferred_element_type=jnp.float32)
        ht_ref[...] = h.astype(jnp.bfloat16).T

    b16 = b_ref[...].astype(jnp.bfloat16)

    x1t = jax.lax.dot_general(ht_ref[...], b16, (((1,), (0,)), ((), ())),
                              preferred_element_type=jnp.float32)
    x1 = x1t.T + b0_ref[...]
    valid = e_total - j * ej
    row_ids = jax.lax.broadcasted_iota(jnp.int32, x1.shape, 0)
    x1 = jnp.where(row_ids < valid, x1, 0.0)
    x1_out_ref[...] = jnp.maximum(x1, 0.0)

    y = jnp.dot(x1, w1_ref[...],
                preferred_element_type=jnp.float32).astype(jnp.bfloat16)
    contrib = jax.lax.dot_general(b16, y, (((1,), (0,)), ((), ())),
                                  preferred_element_type=jnp.float32)

    @pl.when(j == 0)
    def _():
        x0_out_ref[...] = contrib

    @pl.when(j > 0)
    def _():
        x0_out_ref[...] += contrib

    @pl.when(j == nj - 1)
    def _():
        x0_out_ref[...] = jnp.maximum(x0_out_ref[...] + b1_ref[...], 0.0)


def kernel(x_0, incidence_1, W0, W1, bias_0_to_1, bias_1_to_0):
    n, d_in = x_0.shape
    e = incidence_1.shape[1]
    d = W0.shape[1]

    ej = min(384, ((e + 127) // 128) * 128)
    grid = ((e + ej - 1) // ej,)

    out0, out1 = pl.pallas_call(
        functools.partial(_hnhn_block, e_total=e),
        grid=grid,
        in_specs=[
            pl.BlockSpec((n, d_in), lambda j: (0, 0)),
            pl.BlockSpec((n, ej), lambda j: (0, j)),
            pl.BlockSpec((d_in, d), lambda j: (0, 0)),
            pl.BlockSpec((d, d), lambda j: (0, 0)),
            pl.BlockSpec((1, d), lambda j: (0, 0)),
            pl.BlockSpec((1, d), lambda j: (0, 0)),
        ],
        out_specs=[
            pl.BlockSpec((n, d), lambda j: (0, 0)),
            pl.BlockSpec((ej, d), lambda j: (j, 0)),
        ],
        out_shape=[
            jax.ShapeDtypeStruct((n, d), jnp.float32),
            jax.ShapeDtypeStruct((e, d), jnp.float32),
        ],
        scratch_shapes=[pltpu.VMEM((d, n), jnp.bfloat16)],
        compiler_params=pltpu.CompilerParams(
            dimension_semantics=("arbitrary",),
            vmem_limit_bytes=64 * 1024 * 1024,
        ),
    )(x_0, incidence_1, W0, W1, bias_0_to_1, bias_1_to_0)
    return (out0, out1)


# speedup vs baseline: 1.9194x; 1.1074x over previous
import functools

import jax
import jax.numpy as jnp
from jax.experimental import pallas as pl
from jax.experimental.pallas import tpu as pltpu


def _ht_block(x0_ref, w0_ref, ht_ref):
    h = jnp.dot(x0_ref[...].astype(jnp.bfloat16),
                w0_ref[...].astype(jnp.bfloat16),
                preferred_element_type=jnp.float32)
    ht_ref[...] = h.astype(jnp.bfloat16).T


def _hnhn_block(ht_ref, b_ref, w1_ref, b0_ref, b1_ref,
                x0_out_ref, x1_out_ref, *, e_total):
    j = pl.program_id(0)
    nj = pl.num_programs(0)
    ej = x1_out_ref.shape[0]

    b16 = b_ref[...].astype(jnp.bfloat16)

    x1t = jax.lax.dot_general(ht_ref[...], b16, (((1,), (0,)), ((), ())),
                              preferred_element_type=jnp.float32)
    x1 = x1t.T + b0_ref[...]
    valid = e_total - j * ej
    row_ids = jax.lax.broadcasted_iota(jnp.int32, x1.shape, 0)
    x1 = jnp.where(row_ids < valid, x1, 0.0)
    x1_out_ref[...] = jnp.maximum(x1, 0.0)

    y = jnp.dot(x1.astype(jnp.bfloat16), w1_ref[...],
                preferred_element_type=jnp.float32).astype(jnp.bfloat16)

    @pl.when(j == 0)
    def _():
        x0_out_ref[...] = jnp.zeros_like(x0_out_ref)

    x0_out_ref[...] += jax.lax.dot_general(
        b16, y, (((1,), (0,)), ((), ())),
        preferred_element_type=jnp.float32)

    @pl.when(j == nj - 1)
    def _():
        x0_out_ref[...] = jnp.maximum(x0_out_ref[...] + b1_ref[...], 0.0)


def kernel(x_0, incidence_1, W0, W1, bias_0_to_1, bias_1_to_0):
    n, d_in = x_0.shape
    e = incidence_1.shape[1]
    d = W0.shape[1]

    ht = pl.pallas_call(
        _ht_block,
        out_shape=jax.ShapeDtypeStruct((d, n), jnp.bfloat16),
    )(x_0, W0)

    ej = min(512, ((e + 127) // 128) * 128)
    grid = ((e + ej - 1) // ej,)

    out0, out1 = pl.pallas_call(
        functools.partial(_hnhn_block, e_total=e),
        grid=grid,
        in_specs=[
            pl.BlockSpec((d, n), lambda j: (0, 0)),
            pl.BlockSpec((n, ej), lambda j: (0, j)),
            pl.BlockSpec((d, d), lambda j: (0, 0)),
            pl.BlockSpec((1, d), lambda j: (0, 0)),
            pl.BlockSpec((1, d), lambda j: (0, 0)),
        ],
        out_specs=[
            pl.BlockSpec((n, d), lambda j: (0, 0)),
            pl.BlockSpec((ej, d), lambda j: (j, 0)),
        ],
        out_shape=[
            jax.ShapeDtypeStruct((n, d), jnp.float32),
            jax.ShapeDtypeStruct((e, d), jnp.float32),
        ],
        compiler_params=pltpu.CompilerParams(
            dimension_semantics=("arbitrary",),
            vmem_limit_bytes=64 * 1024 * 1024,
        ),
    )(ht, incidence_1, W1, bias_0_to_1, bias_1_to_0)
    return (out0, out1)
